# Initial kernel scaffold; baseline (speedup 1.0000x reference)
import jax
import jax.numpy as jnp
from jax.experimental import pallas as pl


def kernel(weight1, weight2, sig):
    raise NotImplementedError("write your pallas kernel here")


# speedup vs baseline: 20.4713x; 20.4713x over previous
import functools

import jax
import jax.numpy as jnp
import numpy as np
from jax.experimental import pallas as pl
from jax.experimental.pallas import tpu as pltpu

_OBJ0, _OBJ1 = 256, 4096
_K = 410
_POOL = 16
_BLK = 128
_THRESHOLD = 0.1
_LAMDA = 1.0

_A = np.zeros((_OBJ1, _OBJ1 // _POOL), dtype=np.float32)
_A[np.arange(_OBJ1), np.arange(_OBJ1) // _POOL] = 1.0 / 32.0


def _pool_body(w1_ref, w2_ref, a_ref, out_ref):
    s = w1_ref[...] + w2_ref[...]
    out_ref[...] = jax.lax.dot(s, a_ref[...], preferred_element_type=jnp.float32)


def _select_body(e_ref, sig_ref, loss_ref):
    e = e_ref[...]
    bits = jax.lax.bitcast_convert_type(jnp.abs(e), jnp.int32)

    lo = jnp.zeros((_OBJ0, 1), jnp.int32)
    hi = jnp.full((_OBJ0, 1), 0x7F800000, jnp.int32)

    def body(_, carry):
        lo, hi = carry
        mid = lo + ((hi - lo) >> 1)
        cnt = jnp.sum((bits <= mid).astype(jnp.int32), axis=1, keepdims=True)
        take = cnt >= _K
        return jnp.where(take, lo, mid + 1), jnp.where(take, mid, hi)

    lo, hi = jax.lax.fori_loop(0, 31, body, (lo, hi))
    kth = lo

    less = bits < kth
    eq = bits == kth
    cnt_less = jnp.sum(less.astype(jnp.float32), axis=1, keepdims=True)
    cnt_eq = jnp.sum(eq.astype(jnp.float32), axis=1, keepdims=True)
    sum_less = jnp.sum(jnp.where(less, e, 0.0), axis=1, keepdims=True)
    sum_eq = jnp.sum(jnp.where(eq, e, 0.0), axis=1, keepdims=True)
    total = jnp.sum(e, axis=1, keepdims=True)

    need = jnp.float32(_K) - cnt_less
    bottom = sum_less + sum_eq * need / cnt_eq
    pred = (total - bottom) / jnp.float32(_OBJ1)

    sig = sig_ref[...]
    loss = _LAMDA * jnp.sum(jax.nn.relu(_THRESHOLD - sig * pred))
    loss_ref[0, 0] = loss


@functools.partial(jax.jit, static_argnames=("interpret",))
def kernel(weight1, weight2, sig, interpret=False):
    grid = weight1.shape[0] // _BLK
    pooled = pl.pallas_call(
        _pool_body,
        grid=(grid,),
        in_specs=[
            pl.BlockSpec((_BLK, _OBJ1), lambda i: (i, 0)),
            pl.BlockSpec((_BLK, _OBJ1), lambda i: (i, 0)),
            pl.BlockSpec((_OBJ1, _OBJ0), lambda i: (0, 0)),
        ],
        out_specs=pl.BlockSpec((_BLK, _OBJ0), lambda i: (i, 0)),
        out_shape=jax.ShapeDtypeStruct((weight1.shape[0], _OBJ0), jnp.float32),
        interpret=interpret,
    )(weight1, weight2, jnp.asarray(_A))

    extraction = pooled.reshape(_OBJ0, _OBJ1)

    loss = pl.pallas_call(
        _select_body,
        in_specs=[
            pl.BlockSpec((_OBJ0, _OBJ1), lambda: (0, 0)),
            pl.BlockSpec((_OBJ0, 1), lambda: (0, 0)),
        ],
        out_specs=pl.BlockSpec(memory_space=pltpu.SMEM),
        out_shape=jax.ShapeDtypeStruct((1, 1), jnp.float32),
        interpret=interpret,
    )(extraction, sig.reshape(_OBJ0, 1))
    return loss[0, 0]
